# baseline (speedup 1.0000x reference)
import functools

import jax
import jax.numpy as jnp
from jax import lax
from jax.experimental import pallas as pl
from jax.experimental.pallas import tpu as pltpu
from jax.experimental.pallas import tpu_sc as plsc

B = 4096
LSEQ = 50
EMB = 256
OUT = 1000
VOCAB = 100000

NC = 2
NS = 16
LANES = 16
NW = NC * NS
NREG = EMB // LANES

NCHUNKS = 2
BC = B // NCHUNKS
BPW = BC // NW


def _sc_pool_body(src_hbm, table_hbm, out_hbm, idx_v, rows0, rows1, acc_v,
                  sem0, sem1):
    c = lax.axis_index("c")
    s = lax.axis_index("s")
    wid = s * NC + c
    base = wid * BPW

    pltpu.sync_copy(src_hbm.at[pl.ds(base, BPW), :], idx_v)

    pltpu.async_copy(table_hbm.at[idx_v.at[0]], rows0, sem0)
    pltpu.async_copy(table_hbm.at[idx_v.at[1]], rows1, sem1)

    def process(rows, row_i):
        def jbody(j, accs):
            return tuple(accs[k] + rows[j, pl.ds(k * LANES, LANES)]
                         for k in range(NREG))
        zero = jnp.zeros((LANES,), jnp.float32)
        accs = lax.fori_loop(0, LSEQ, jbody, (zero,) * NREG)
        for k in range(NREG):
            acc_v[row_i, pl.ds(k * LANES, LANES)] = accs[k]

    def obody(i, carry):
        r0 = 2 * i
        pltpu.make_async_copy(table_hbm.at[idx_v.at[r0]], rows0, sem0).wait()
        process(rows0, r0)

        @pl.when(r0 + 2 < BPW)
        def _():
            pltpu.async_copy(table_hbm.at[idx_v.at[r0 + 2]], rows0, sem0)

        pltpu.make_async_copy(table_hbm.at[idx_v.at[r0 + 1]], rows1,
                              sem1).wait()
        process(rows1, r0 + 1)

        @pl.when(r0 + 3 < BPW)
        def _():
            pltpu.async_copy(table_hbm.at[idx_v.at[r0 + 3]], rows1, sem1)

        return carry

    lax.fori_loop(0, BPW // 2, obody, 0)

    pltpu.sync_copy(acc_v, out_hbm.at[pl.ds(base, BPW), :])


def _sc_pool(src_chunk, table):
    mesh = plsc.VectorSubcoreMesh(core_axis_name="c", subcore_axis_name="s")
    f = pl.kernel(
        _sc_pool_body,
        out_type=jax.ShapeDtypeStruct((BC, EMB), jnp.float32),
        mesh=mesh,
        scratch_types=[
            pltpu.VMEM((BPW, LSEQ), jnp.int32),
            pltpu.VMEM((LSEQ, EMB), jnp.float32),
            pltpu.VMEM((LSEQ, EMB), jnp.float32),
            pltpu.VMEM((BPW, EMB), jnp.float32),
            pltpu.SemaphoreType.DMA,
            pltpu.SemaphoreType.DMA,
        ],
    )
    return f(src_chunk, table)


def _tc_finish_body(emb_ref, src_ref, w_ref, b_ref, out_ref):
    x = emb_ref[...]
    cnt = jnp.sum((src_ref[...] != 0).astype(jnp.float32), axis=1,
                  keepdims=True)
    x = x / cnt
    logits = lax.dot_general(x, w_ref[...], (((1,), (1,)), ((), ())),
                             preferred_element_type=jnp.float32,
                             precision=lax.Precision.HIGHEST)
    logits = logits + b_ref[...]
    m = jnp.max(logits, axis=-1, keepdims=True)
    sh = logits - m
    lse = jnp.log(jnp.sum(jnp.exp(sh), axis=-1, keepdims=True))
    out_ref[...] = sh - lse


def _tc_finish(emb_sum, src_chunk, W, b2d):
    BB = 512
    return pl.pallas_call(
        _tc_finish_body,
        grid=(BC // BB,),
        in_specs=[
            pl.BlockSpec((BB, EMB), lambda i: (i, 0)),
            pl.BlockSpec((BB, LSEQ), lambda i: (i, 0)),
            pl.BlockSpec((OUT, EMB), lambda i: (0, 0)),
            pl.BlockSpec((1, OUT), lambda i: (0, 0)),
        ],
        out_specs=pl.BlockSpec((BB, OUT), lambda i: (i, 0)),
        out_shape=jax.ShapeDtypeStruct((BC, OUT), jnp.float32),
    )(emb_sum, src_chunk, W, b2d)


def kernel(src, table, W, b):
    b2d = b.reshape(1, OUT)
    outs = []
    for k in range(NCHUNKS):
        src_k = lax.slice_in_dim(src, k * BC, (k + 1) * BC, axis=0)
        emb_k = _sc_pool(src_k, table)
        outs.append(_tc_finish(emb_k, src_k, W, b2d))
    return jnp.concatenate(outs, axis=0)


# speedup vs baseline: 1.0648x; 1.0648x over previous
import functools

import jax
import jax.numpy as jnp
from jax import lax
from jax.experimental import pallas as pl
from jax.experimental.pallas import tpu as pltpu
from jax.experimental.pallas import tpu_sc as plsc

B = 4096
LSEQ = 50
EMB = 256
OUT = 1000
VOCAB = 100000

NC = 2
NS = 16
LANES = 16
NW = NC * NS
NREG = EMB // LANES

NCHUNKS = 2
BC = B // NCHUNKS
BPW = BC // NW


def _sc_pool_body(src_hbm, table_hbm, out_hbm, idx_v, rows0, rows1, acc_v,
                  sem0, sem1):
    c = lax.axis_index("c")
    s = lax.axis_index("s")
    wid = s * NC + c
    base = wid * BPW

    pltpu.sync_copy(src_hbm.at[pl.ds(base, BPW), :], idx_v)

    pltpu.async_copy(table_hbm.at[idx_v.at[0]], rows0, sem0)
    pltpu.async_copy(table_hbm.at[idx_v.at[1]], rows1, sem1)

    def process(rows, row_i):
        def jbody(j, accs):
            return tuple(accs[k] + rows[j, pl.ds(k * LANES, LANES)]
                         for k in range(NREG))
        zero = jnp.zeros((LANES,), jnp.float32)
        accs = lax.fori_loop(0, LSEQ, jbody, (zero,) * NREG)
        for k in range(NREG):
            acc_v[row_i, pl.ds(k * LANES, LANES)] = accs[k]

    def obody(i, carry):
        r0 = 2 * i
        pltpu.make_async_copy(table_hbm.at[idx_v.at[r0]], rows0, sem0).wait()
        process(rows0, r0)

        @pl.when(r0 + 2 < BPW)
        def _():
            pltpu.async_copy(table_hbm.at[idx_v.at[r0 + 2]], rows0, sem0)

        pltpu.make_async_copy(table_hbm.at[idx_v.at[r0 + 1]], rows1,
                              sem1).wait()
        process(rows1, r0 + 1)

        @pl.when(r0 + 3 < BPW)
        def _():
            pltpu.async_copy(table_hbm.at[idx_v.at[r0 + 3]], rows1, sem1)

        return carry

    lax.fori_loop(0, BPW // 2, obody, 0)

    pltpu.sync_copy(acc_v, out_hbm.at[pl.ds(base, BPW), :])


def _sc_pool(src_chunk, table):
    mesh = plsc.VectorSubcoreMesh(core_axis_name="c", subcore_axis_name="s")
    f = pl.kernel(
        _sc_pool_body,
        out_type=jax.ShapeDtypeStruct((BC, EMB), jnp.float32),
        mesh=mesh,
        scratch_types=[
            pltpu.VMEM((BPW, LSEQ), jnp.int32),
            pltpu.VMEM((LSEQ, EMB), jnp.float32),
            pltpu.VMEM((LSEQ, EMB), jnp.float32),
            pltpu.VMEM((BPW, EMB), jnp.float32),
            pltpu.SemaphoreType.DMA,
            pltpu.SemaphoreType.DMA,
        ],
    )
    return f(src_chunk, table)


def _tc_finish_body(emb_ref, src_ref, w_ref, b_ref, out_ref):
    x = emb_ref[...]
    cnt = jnp.sum((src_ref[...] != 0).astype(jnp.float32), axis=1,
                  keepdims=True)
    x = x / cnt
    logits = lax.dot_general(x, w_ref[...], (((1,), (1,)), ((), ())),
                             preferred_element_type=jnp.float32,
                             precision=lax.Precision.HIGHEST)
    logits = logits + b_ref[...]
    m = jnp.max(logits, axis=-1, keepdims=True)
    sh = logits - m
    lse = jnp.log(jnp.sum(jnp.exp(sh), axis=-1, keepdims=True))
    out_ref[...] = sh - lse


def _tc_finish_body_aliased(emb_ref, src_ref, w_ref, b_ref, prev_ref,
                            out_ref):
    del prev_ref
    _tc_finish_body(emb_ref, src_ref, w_ref, b_ref, out_ref)


TC_BB = 512


def _tc_finish(emb_sum, src_chunk, W, b2d, k, prev):
    nblk = BC // TC_BB
    blk0 = k * nblk
    in_specs = [
        pl.BlockSpec((TC_BB, EMB), lambda i: (i, 0)),
        pl.BlockSpec((TC_BB, LSEQ), lambda i: (i, 0)),
        pl.BlockSpec((OUT, EMB), lambda i: (0, 0)),
        pl.BlockSpec((1, OUT), lambda i: (0, 0)),
    ]
    args = [emb_sum, src_chunk, W, b2d]
    body = _tc_finish_body
    aliases = {}
    if prev is not None:
        in_specs.append(pl.BlockSpec(memory_space=pl.ANY))
        args.append(prev)
        body = _tc_finish_body_aliased
        aliases = {4: 0}
    return pl.pallas_call(
        body,
        grid=(nblk,),
        in_specs=in_specs,
        out_specs=pl.BlockSpec((TC_BB, OUT), lambda i, b0=blk0: (i + b0, 0)),
        out_shape=jax.ShapeDtypeStruct((B, OUT), jnp.float32),
        input_output_aliases=aliases,
    )(*args)


def kernel(src, table, W, b):
    b2d = b.reshape(1, OUT)
    out = None
    for k in range(NCHUNKS):
        src_k = lax.slice_in_dim(src, k * BC, (k + 1) * BC, axis=0)
        emb_k = _sc_pool(src_k, table)
        out = _tc_finish(emb_k, src_k, W, b2d, k, out)
    return out
